# Initial kernel scaffold; baseline (speedup 1.0000x reference)
import jax
import jax.numpy as jnp
from jax.experimental import pallas as pl


def kernel(t, h, edge_index, batch_size, W, b, gamma, beta):
    raise NotImplementedError("write your pallas kernel here")


# speedup vs baseline: 20.0896x; 20.0896x over previous
import functools

import jax
import jax.numpy as jnp
from jax import lax
from jax.experimental import pallas as pl
from jax.experimental.pallas import tpu as pltpu
from jax.experimental.pallas import tpu_sc as plsc

N = 10000
E = 320000
D = 128

NC = 2
NS = 16
NW = NC * NS

K = 128
CH = 79
EPT = CH * K
E2 = EPT * NW
XP = N + 8

DEGP = 10240
DROW = DEGP // NS

NP = 10240
RPT = NP // NS
RZ = 64

_mesh = plsc.VectorSubcoreMesh(core_axis_name="c", subcore_axis_name="s")


@functools.partial(
    pl.kernel,
    out_type=jax.ShapeDtypeStruct((NC, DEGP), jnp.float32),
    mesh=_mesh,
    scratch_types=[
        pltpu.VMEM((CH, K), jnp.int32),
        pltpu.VMEM((K,), jnp.float32),
        pltpu.VMEM((DROW,), jnp.float32),
        pltpu.VMEM_SHARED((DEGP,), jnp.float32),
    ],
)
def _sc_deg(dst_hbm, out_hbm, didx, ones_v, zbuf, deg_s):
    c = lax.axis_index("c")
    s = lax.axis_index("s")
    wid = c * NS + s
    pltpu.sync_copy(dst_hbm.at[wid], didx)
    for j in range(K // 16):
        ones_v[pl.ds(j * 16, 16)] = jnp.ones((16,), jnp.float32)
    for j in range(DROW // 16):
        zbuf[pl.ds(j * 16, 16)] = jnp.zeros((16,), jnp.float32)
    pltpu.sync_copy(zbuf, deg_s.at[pl.ds(s * DROW, DROW)])
    plsc.subcore_barrier()

    def body(i, carry):
        pltpu.sync_copy(ones_v, deg_s.at[didx.at[i]], add=True)
        return carry

    lax.fori_loop(0, CH, body, 0)
    plsc.subcore_barrier()
    pltpu.sync_copy(deg_s.at[pl.ds(s * DROW, DROW)],
                    out_hbm.at[c, pl.ds(s * DROW, DROW)])


@functools.partial(
    pl.kernel,
    out_type=jax.ShapeDtypeStruct((NC, NP, D), jnp.float32),
    mesh=_mesh,
    scratch_types=[
        pltpu.VMEM((CH, K), jnp.int32),
        pltpu.VMEM((CH, K), jnp.int32),
        pltpu.VMEM((K, D), jnp.float32),
        pltpu.VMEM((RZ, D), jnp.float32),
        pltpu.VMEM_SHARED((NP, D), jnp.float32),
    ],
)
def _sc_msg(xs_hbm, src_hbm, dst_hbm, zacc_hbm, out_hbm,
            sidx, didx, rows, zbuf, acc_s):
    c = lax.axis_index("c")
    s = lax.axis_index("s")
    wid = c * NS + s
    pltpu.sync_copy(src_hbm.at[wid], sidx)
    pltpu.sync_copy(dst_hbm.at[wid], didx)
    pltpu.sync_copy(zacc_hbm, zbuf)
    for q in range(RPT // RZ):
        pltpu.sync_copy(zbuf, acc_s.at[pl.ds(s * RPT + q * RZ, RZ)])
    plsc.subcore_barrier()

    def body(i, carry):
        pltpu.sync_copy(xs_hbm.at[sidx.at[i]], rows)
        pltpu.sync_copy(rows, acc_s.at[didx.at[i]], add=True)
        return carry

    lax.fori_loop(0, CH, body, 0)
    plsc.subcore_barrier()
    for q in range(RPT // RZ):
        r0 = s * RPT + q * RZ
        pltpu.sync_copy(acc_s.at[pl.ds(r0, RZ)], out_hbm.at[c, pl.ds(r0, RZ)])


def _tc_mm_body(h_ref, w_ref, degt_ref, xs_ref):
    deg = degt_ref[:, 0:1] + degt_ref[:, 1:2] + 1.0
    dinv = lax.rsqrt(deg)
    x = jnp.dot(h_ref[:, :], w_ref[:, :], preferred_element_type=jnp.float32)
    xs_ref[:, :] = x * dinv


_tc_mm = pl.pallas_call(
    _tc_mm_body,
    out_shape=jax.ShapeDtypeStruct((N, D), jnp.float32),
)


def _tc_fin_body(accp_ref, xs_ref, degt_ref, b_ref, g_ref, be_ref, out_ref):
    deg = degt_ref[:, 0:1] + degt_ref[:, 1:2] + 1.0
    dinv = lax.rsqrt(deg)
    agg = accp_ref[0] + accp_ref[1] + xs_ref[:, :]
    o = agg * dinv + b_ref[:, :]
    mu = jnp.mean(o, axis=-1, keepdims=True)
    cen = o - mu
    var = jnp.mean(cen * cen, axis=-1, keepdims=True)
    y = cen * lax.rsqrt(var + 1e-5) * g_ref[:, :] + be_ref[:, :]
    out_ref[:, :] = jnp.tanh(y)


_tc_fin = pl.pallas_call(
    _tc_fin_body,
    out_shape=jax.ShapeDtypeStruct((N, D), jnp.float32),
)


def kernel(t, h, edge_index, batch_size, W, b, gamma, beta):
    if h.ndim == 1:
        h = h[None, :]
    pad = jnp.full((E2 - E,), N, dtype=edge_index.dtype)
    src = jnp.concatenate([edge_index[0], pad]).reshape(NW, CH, K)
    dst = jnp.concatenate([edge_index[1], pad]).reshape(NW, CH, K)
    zacc = jnp.zeros((RZ, D), jnp.float32)

    degp = _sc_deg(dst)
    degt = jnp.stack([degp[0, :N], degp[1, :N]], axis=1)
    xs = _tc_mm(h, W, degt)
    xs_p = jnp.concatenate([xs, jnp.zeros((XP - N, D), jnp.float32)])
    accp = _sc_msg(xs_p, src, dst, zacc)[:, :N]
    dh = _tc_fin(accp, xs, degt, b[None, :], gamma[None, :], beta[None, :])
    return (dh, jnp.zeros_like(edge_index), jnp.zeros_like(batch_size))
